# Initial kernel scaffold; baseline (speedup 1.0000x reference)
import jax
import jax.numpy as jnp
from jax.experimental import pallas as pl


def kernel(x, edge_index, edge_attr, batch, emb, W1, as1, ad1, W2, as2, ad2, Wm1, bm1, Wm2, bm2):
    raise NotImplementedError("write your pallas kernel here")


# speedup vs baseline: 14.4982x; 14.4982x over previous
import functools

import jax
import jax.numpy as jnp
from jax import lax
from jax.experimental import pallas as pl
from jax.experimental.pallas import tpu as pltpu
from jax.experimental.pallas import tpu_sc as plsc

N = 10000
E = 320000
D = 128
V = 100000
B = 16

NC = 2
NS = 16
NW = NC * NS

NP = 10240
RPW = NP // NW
CH = 128
EP = 331776
EPW = EP // NW
NCHUNK = EPW // CH
DUMMY = N + 64
RB = 2048

_mesh = plsc.VectorSubcoreMesh(core_axis_name="c", subcore_axis_name="s")
_sc_params = pltpu.CompilerParams(needs_layout_passes=False)


@functools.partial(
    pl.kernel,
    out_type=jax.ShapeDtypeStruct((NP, D), jnp.float32),
    mesh=_mesh,
    compiler_params=_sc_params,
    scratch_types=[
        pltpu.VMEM((RPW,), jnp.int32),
        pltpu.VMEM((RPW, D), jnp.float32),
        pltpu.SemaphoreType.DMA,
    ],
)
def _emb_gather(emb_hbm, idx_hbm, out_hbm, idx_v, rows_v, sem):
    c = lax.axis_index("c")
    s = lax.axis_index("s")
    wid = s * NC + c
    base = wid * RPW
    pltpu.sync_copy(idx_hbm.at[pl.ds(base, RPW)], idx_v)
    for j in range(RPW // 80):
        pltpu.async_copy(
            emb_hbm.at[idx_v.at[pl.ds(j * 80, 80)]],
            rows_v.at[pl.ds(j * 80, 80)],
            sem,
        ).wait()
    pltpu.sync_copy(rows_v, out_hbm.at[pl.ds(base, RPW)])


@functools.partial(
    pl.kernel,
    out_type=[
        jax.ShapeDtypeStruct((NC, NP, D), jnp.float32),
        jax.ShapeDtypeStruct((NC, NS, NP), jnp.float32),
    ],
    mesh=_mesh,
    compiler_params=_sc_params,
    scratch_types=[
        pltpu.VMEM((NP,), jnp.float32),
        pltpu.VMEM((NP,), jnp.float32),
        pltpu.VMEM((NP + 16,), jnp.float32),
        pltpu.VMEM((CH,), jnp.int32),
        pltpu.VMEM((CH,), jnp.int32),
        pltpu.VMEM((CH,), jnp.float32),
        pltpu.VMEM((CH,), jnp.float32),
        pltpu.VMEM((CH, D), jnp.float32),
        pltpu.VMEM_SHARED((NP, D), jnp.float32),
        pltpu.SemaphoreType.DMA,
    ],
)
def _edge_pass(H_hbm, ss_hbm, sd_hbm, src_hbm, dst_hbm, ea_hbm,
               acc_hbm, den_hbm,
               ss_v, sd_v, den_v, srcv, dstv, eav, exv, rows_v, acc_sh, sem):
    c = lax.axis_index("c")
    s = lax.axis_index("s")
    wid = s * NC + c

    zero16 = jnp.zeros((16,), jnp.float32)

    def zrow(j, _):
        for dc in range(D // 16):
            rows_v[j, pl.ds(dc * 16, 16)] = zero16
        return 0

    lax.fori_loop(0, CH, zrow, 0)
    for k in range(RPW // CH):
        pltpu.sync_copy(rows_v, acc_sh.at[pl.ds(s * RPW + k * CH, CH)])

    def zden(j, _):
        den_v[pl.ds(j * 16, 16)] = zero16
        return 0

    lax.fori_loop(0, (NP + 16) // 16, zden, 0)

    pltpu.sync_copy(ss_hbm, ss_v)
    pltpu.sync_copy(sd_hbm, sd_v)
    plsc.subcore_barrier()

    e0 = wid * EPW

    def chunk(k, _):
        eb = e0 + k * CH
        pltpu.sync_copy(src_hbm.at[pl.ds(eb, CH)], srcv)
        pltpu.sync_copy(dst_hbm.at[pl.ds(eb, CH)], dstv)
        pltpu.sync_copy(ea_hbm.at[pl.ds(eb, CH)], eav)
        cp = pltpu.async_copy(H_hbm.at[srcv], rows_v, sem)

        def exb(j, _):
            sl = pl.ds(j * 16, 16)
            a = plsc.load_gather(ss_v, [srcv[sl]]) + \
                plsc.load_gather(sd_v, [dstv[sl]])
            a = jnp.where(a >= 0.0, a, 0.2 * a) * eav[sl]
            exv[sl] = jnp.exp(a)
            return 0

        lax.fori_loop(0, CH // 16, exb, 0)
        cp.wait()

        lane = lax.broadcasted_iota(jnp.int32, (16,), 0)

        def scale(j16, _):
            base = j16 * 16
            ex16 = exv[pl.ds(base, 16)]
            d16 = dstv[pl.ds(base, 16)]
            for l in range(16):
                e = ex16[l]
                dj = d16[l]
                plsc.addupdate(den_v.at[pl.ds(dj, 16)],
                               jnp.where(lane == 0, e, 0.0))
                jj = base + l
                for dc in range(D // 16):
                    sl = pl.ds(dc * 16, 16)
                    rows_v[jj, sl] = rows_v[jj, sl] * e
            return 0

        lax.fori_loop(0, CH // 16, scale, 0)
        pltpu.sync_copy(rows_v, acc_sh.at[dstv], add=True)
        return 0

    lax.fori_loop(0, NCHUNK, chunk, 0)
    plsc.subcore_barrier()

    pltpu.sync_copy(den_v.at[pl.ds(0, NP)], den_hbm.at[c, s])
    pltpu.sync_copy(acc_sh.at[pl.ds(s * RPW, RPW)],
                    acc_hbm.at[c, pl.ds(s * RPW, RPW)])


def _tc1_body(h_ref, W_ref, asrc_ref, adst_ref, H_ref, ss_ref, sd_ref):
    Hb = lax.dot_general(h_ref[...], W_ref[...], (((1,), (1,)), ((), ())),
                         preferred_element_type=jnp.float32)
    H_ref[...] = Hb
    ss_ref[...] = jnp.sum(Hb * asrc_ref[...][None, :], axis=1)
    sd_ref[...] = jnp.sum(Hb * adst_ref[...][None, :], axis=1)


_tc1 = pl.pallas_call(
    _tc1_body,
    grid=(NP // RB,),
    in_specs=[
        pl.BlockSpec((RB, D), lambda i: (i, 0)),
        pl.BlockSpec((D, D), lambda i: (0, 0)),
        pl.BlockSpec((D,), lambda i: (0,)),
        pl.BlockSpec((D,), lambda i: (0,)),
    ],
    out_specs=[
        pl.BlockSpec((RB, D), lambda i: (i, 0)),
        pl.BlockSpec((RB,), lambda i: (i,)),
        pl.BlockSpec((RB,), lambda i: (i,)),
    ],
    out_shape=[
        jax.ShapeDtypeStruct((NP, D), jnp.float32),
        jax.ShapeDtypeStruct((NP,), jnp.float32),
        jax.ShapeDtypeStruct((NP,), jnp.float32),
    ],
)


def _tc2_body(acc_ref, den_ref, W_ref, asrc_ref, adst_ref,
              H_ref, ss_ref, sd_ref):
    den = jnp.sum(den_ref[...], axis=(0, 1))
    h = (acc_ref[0] + acc_ref[1]) / (den + 1e-16)[:, None]
    h = jnp.maximum(h, 0.0)
    Hb = lax.dot_general(h, W_ref[...], (((1,), (1,)), ((), ())),
                         preferred_element_type=jnp.float32)
    H_ref[...] = Hb
    ss_ref[...] = jnp.sum(Hb * asrc_ref[...][None, :], axis=1)
    sd_ref[...] = jnp.sum(Hb * adst_ref[...][None, :], axis=1)


_tc2 = pl.pallas_call(
    _tc2_body,
    grid=(NP // RB,),
    in_specs=[
        pl.BlockSpec((NC, RB, D), lambda i: (0, i, 0)),
        pl.BlockSpec((NC, NS, RB), lambda i: (0, 0, i)),
        pl.BlockSpec((D, D), lambda i: (0, 0)),
        pl.BlockSpec((D,), lambda i: (0,)),
        pl.BlockSpec((D,), lambda i: (0,)),
    ],
    out_specs=[
        pl.BlockSpec((RB, D), lambda i: (i, 0)),
        pl.BlockSpec((RB,), lambda i: (i,)),
        pl.BlockSpec((RB,), lambda i: (i,)),
    ],
    out_shape=[
        jax.ShapeDtypeStruct((NP, D), jnp.float32),
        jax.ShapeDtypeStruct((NP,), jnp.float32),
        jax.ShapeDtypeStruct((NP,), jnp.float32),
    ],
)


def _tc3_body(acc_ref, den_ref, batch_ref, Wm1_ref, bm1_ref, Wm2_ref, bm2_ref,
              out_ref, g_acc, cnt_acc):
    i = pl.program_id(0)

    @pl.when(i == 0)
    def _():
        g_acc[...] = jnp.zeros_like(g_acc)
        cnt_acc[...] = jnp.zeros_like(cnt_acc)

    den = jnp.sum(den_ref[...], axis=(0, 1))
    h = jnp.maximum((acc_ref[0] + acc_ref[1]) / (den + 1e-16)[:, None], 0.0)
    rows = lax.broadcasted_iota(jnp.int32, (B, RB), 0)
    onehot = jnp.where(batch_ref[...][None, :] == rows, 1.0, 0.0)
    g_acc[...] += jnp.dot(onehot, h, preferred_element_type=jnp.float32)
    cnt_acc[...] += jnp.sum(onehot, axis=1, keepdims=True)

    @pl.when(i == pl.num_programs(0) - 1)
    def _():
        g = g_acc[...] / jnp.maximum(cnt_acc[...], 1.0)
        z = lax.dot_general(g, Wm1_ref[...], (((1,), (1,)), ((), ())),
                            preferred_element_type=jnp.float32)
        z = jnp.maximum(z + bm1_ref[...][None, :], 0.0)
        o = lax.dot_general(z, Wm2_ref[...], (((1,), (1,)), ((), ())),
                            preferred_element_type=jnp.float32)
        out_ref[...] = o + bm2_ref[...][None, :]


_tc3 = pl.pallas_call(
    _tc3_body,
    grid=(NP // RB,),
    in_specs=[
        pl.BlockSpec((NC, RB, D), lambda i: (0, i, 0)),
        pl.BlockSpec((NC, NS, RB), lambda i: (0, 0, i)),
        pl.BlockSpec((RB,), lambda i: (i,)),
        pl.BlockSpec((D // 2, D), lambda i: (0, 0)),
        pl.BlockSpec((D // 2,), lambda i: (0,)),
        pl.BlockSpec((D, D // 2), lambda i: (0, 0)),
        pl.BlockSpec((D,), lambda i: (0,)),
    ],
    out_specs=pl.BlockSpec((B, D), lambda i: (0, 0)),
    out_shape=jax.ShapeDtypeStruct((B, D), jnp.float32),
    scratch_shapes=[
        pltpu.VMEM((B, D), jnp.float32),
        pltpu.VMEM((B, D), jnp.float32),
    ],
)


def kernel(x, edge_index, edge_attr, batch, emb,
           W1, as1, ad1, W2, as2, ad2, Wm1, bm1, Wm2, bm2):
    x_p = jnp.zeros((NP,), jnp.int32).at[:N].set(x.astype(jnp.int32))
    loop = jnp.arange(N, dtype=jnp.int32)
    src = jnp.concatenate([edge_index[0].astype(jnp.int32), loop])
    dst = jnp.concatenate([edge_index[1].astype(jnp.int32), loop])
    ea = jnp.concatenate([edge_attr, jnp.ones((N,), jnp.float32)])
    src_p = jnp.zeros((EP,), jnp.int32).at[:E + N].set(src)
    dst_p = jnp.full((EP,), DUMMY, jnp.int32).at[:E + N].set(dst)
    ea_p = jnp.zeros((EP,), jnp.float32).at[:E + N].set(ea)
    batch_p = jnp.full((NP,), B, jnp.int32).at[:N].set(batch.astype(jnp.int32))
    Wm2_p = jnp.zeros((D, D // 2), jnp.float32).at[:2].set(Wm2)
    bm2_p = jnp.zeros((D,), jnp.float32).at[:2].set(bm2)

    h = _emb_gather(emb, x_p)
    H1, ss1, sd1 = _tc1(h, W1, as1, ad1)
    acc1, den1 = _edge_pass(H1, ss1, sd1, src_p, dst_p, ea_p)
    H2, ss2, sd2 = _tc2(acc1, den1, W2, as2, ad2)
    acc2, den2 = _edge_pass(H2, ss2, sd2, src_p, dst_p, ea_p)
    out = _tc3(acc2, den2, batch_p, Wm1, bm1, Wm2_p, bm2_p)
    return out[:, :2]
